# baseline (speedup 1.0000x reference)
import functools

import jax
import jax.numpy as jnp
from jax.experimental import pallas as pl

BM = 512


def _matmul_kernel(x_ref, w_ref, o_ref):
    o_ref[...] = jax.lax.dot_general(
        x_ref[...],
        w_ref[...],
        (((0,), (1,)), ((), ())),
        preferred_element_type=jnp.float32,
    ).astype(jnp.bfloat16)


@functools.partial(jax.jit, static_argnames=())
def kernel(input, weight):
    m, k = weight.shape
    _, n = input.shape
    out_t = pl.pallas_call(
        _matmul_kernel,
        grid=(m // BM,),
        in_specs=[
            pl.BlockSpec((k, n), lambda i: (0, 0)),
            pl.BlockSpec((BM, k), lambda i: (i, 0)),
        ],
        out_specs=pl.BlockSpec((n, BM), lambda i: (0, i)),
        out_shape=jax.ShapeDtypeStruct((n, m), jnp.bfloat16),
    )(input, weight)
    return out_t.T.astype(jnp.float32)


# speedup vs baseline: 1.0764x; 1.0764x over previous
import functools

import jax
import jax.numpy as jnp
from jax.experimental import pallas as pl

BM = 512


def _matmul_kernel(x_ref, w_ref, o_ref):
    o_ref[...] = jax.lax.dot_general(
        x_ref[...],
        w_ref[...],
        (((0,), (1,)), ((), ())),
        preferred_element_type=jnp.float32,
    )


@functools.partial(jax.jit, static_argnames=())
def kernel(input, weight):
    m, k = weight.shape
    _, n = input.shape
    out_t = pl.pallas_call(
        _matmul_kernel,
        grid=(m // BM,),
        in_specs=[
            pl.BlockSpec((k, n), lambda i: (0, 0)),
            pl.BlockSpec((BM, k), lambda i: (i, 0)),
        ],
        out_specs=pl.BlockSpec((n, BM), lambda i: (0, i)),
        out_shape=jax.ShapeDtypeStruct((n, m), jnp.float32),
    )(input, weight)
    return out_t.T
